# baseline (speedup 1.0000x reference)
import jax
import jax.numpy as jnp
from jax.experimental import pallas as pl


def _add_kernel(x_ref, t_ref, o_ref):
    o_ref[...] = x_ref[...] + t_ref[...][None, :, :]


def kernel(x, table):
    B, S, D = x.shape
    TS = 128
    grid = (S // TS,)
    return pl.pallas_call(
        _add_kernel,
        grid=grid,
        in_specs=[
            pl.BlockSpec((B, TS, D), lambda s: (0, s, 0)),
            pl.BlockSpec((TS, D), lambda s: (s, 0)),
        ],
        out_specs=pl.BlockSpec((B, TS, D), lambda s: (0, s, 0)),
        out_shape=jax.ShapeDtypeStruct((B, S, D), x.dtype),
    )(x, table[:S])
